# baseline (speedup 1.0000x reference)
import functools

import jax
import jax.numpy as jnp
from jax import lax
from jax.experimental import pallas as pl
from jax.experimental.pallas import tpu as pltpu
from jax.experimental.pallas import tpu_sc as plsc

_EPS = 1e-12
_NC = 2
_NS = 16
_NW = _NC * _NS
_LANES = 16
_CHUNK = 128
_TTPAD = _CHUNK + _LANES
_UNROLL = 16


def _lane_shuffle(v, idx):
    dnums = lax.GatherDimensionNumbers(
        offset_dims=(), collapsed_slice_dims=(0,), start_index_map=(0,))
    return lax.gather(v, idx[:, None], dnums, slice_sizes=(1,),
                      mode=lax.GatherScatterMode.PROMISE_IN_BOUNDS)


def _allsum(v):
    lane = lax.iota(jnp.int32, _LANES)
    for stride in (1, 2, 4, 8):
        v = v + _lane_shuffle(v, lax.bitwise_xor(lane, stride))
    return v


def _rsqrt(x):
    i = plsc.bitcast(x, jnp.int32)
    i = 0x5F3759DF - lax.shift_right_arithmetic(i, 1)
    y = plsc.bitcast(i, jnp.float32)
    for _ in range(2):
        y = y * (1.5 - 0.5 * x * y * y)
    return y


def _make_sc_kernel(n_tokens, emb, seq):
    per_w = n_tokens // _NW
    n_chunks = per_w // _CHUNK
    n2 = n_chunks // 2
    n_sub = emb // _LANES
    mesh = plsc.VectorSubcoreMesh(core_axis_name="c", subcore_axis_name="s")

    @functools.partial(
        pl.kernel,
        mesh=mesh,
        compiler_params=pltpu.CompilerParams(needs_layout_passes=False),
        out_type=jax.ShapeDtypeStruct((n_tokens, emb), jnp.float32),
        scratch_types=[
            pltpu.VMEM((n_chunks, _CHUNK), jnp.int32),
            pltpu.VMEM((n_chunks, _CHUNK), jnp.int32),
            pltpu.VMEM_SHARED((seq + _CHUNK, emb), jnp.float32),
            pltpu.VMEM((_CHUNK, emb), jnp.float32),
            pltpu.VMEM((_CHUNK, emb), jnp.float32),
            pltpu.VMEM((_CHUNK, emb), jnp.float32),
            pltpu.VMEM((_CHUNK, emb), jnp.float32),
            pltpu.VMEM((3, emb), jnp.float32),
            pltpu.VMEM((_LANES, _LANES), jnp.float32),
            pltpu.VMEM((_LANES, _LANES), jnp.float32),
            pltpu.VMEM((per_w // _CHUNK, _CHUNK), jnp.int32),
            pltpu.SemaphoreType.DMA,
            pltpu.SemaphoreType.DMA,
            pltpu.SemaphoreType.DMA,
            pltpu.SemaphoreType.DMA,
            pltpu.SemaphoreType.DMA,
        ],
    )
    def sc_kernel(wid_hbm, tt_hbm, word_hbm, pos_hbm, cst_hbm, pix_hbm,
                  out_hbm, ids_v, tt_v, pos_v, row0, row1, ob0, ob1, cst_v,
                  svm, qvm, pix_v, sw0, sw1, so0, so1, spa):
        wid = lax.axis_index("s") * _NC + lax.axis_index("c")
        base = wid * per_w
        pltpu.sync_copy(cst_hbm, cst_v)
        sid = lax.axis_index("s")

        @pl.when(sid == 0)
        def _():
            pltpu.sync_copy(pos_hbm, pos_v)

        plsc.subcore_barrier()
        pltpu.sync_copy(wid_hbm.at[wid], ids_v)
        pltpu.sync_copy(tt_hbm.at[wid], tt_v)
        pltpu.sync_copy(pix_hbm, pix_v)
        gs = [cst_v[0, pl.ds(k * _LANES, _LANES)] for k in range(n_sub)]
        bs = [cst_v[1, pl.ds(k * _LANES, _LANES)] for k in range(n_sub)]
        tds = [cst_v[2, pl.ds(k * _LANES, _LANES)] for k in range(n_sub)]

        rows = (row0, row1)
        obs = (ob0, ob1)
        sws = (sw0, sw1)
        sos = (so0, so1)

        def start_gather(ci, b):
            pltpu.make_async_copy(
                word_hbm.at[ids_v.at[ci]], rows[b], sws[b]).start()

        def wait_gather(ci, b):
            pltpu.make_async_copy(
                word_hbm.at[ids_v.at[ci]], rows[b], sws[b]).wait()

        def wait_writeback(b):
            pltpu.make_async_copy(
                obs[b], out_hbm.at[pl.ds(base, _CHUNK)], sos[b]).wait()

        iota16 = lax.iota(jnp.int32, _LANES)

        def pos_add(ci, b):
            pltpu.async_copy(
                pos_v.at[pix_v.at[ci]], rows[b], spa, add=True).wait()

        def compute(ci, b):
            rv, ov = rows[b], obs[b]
            inv_n = 1.0 / emb

            def tok_body(g, carry):
                t0 = g * _UNROLL
                ttw = tt_v[ci, pl.ds(t0, _LANES)].astype(jnp.float32)
                for j in range(_UNROLL):
                    t = t0 + j
                    ttf = _lane_shuffle(ttw, jnp.full((_LANES,), j, jnp.int32))
                    regs = [rv[t, pl.ds(k * _LANES, _LANES)] + ttf * tds[k]
                            for k in range(n_sub)]
                    sv = regs[0]
                    qv = regs[0] * regs[0]
                    for k in range(1, n_sub):
                        sv = sv + regs[k]
                        qv = qv + regs[k] * regs[k]
                    svm[j] = sv
                    qvm[j] = qv
                    for k in range(n_sub):
                        ov[t, pl.ds(k * _LANES, _LANES)] = regs[k]
                tot_s = plsc.load_gather(svm, [iota16, jnp.zeros((_LANES,), jnp.int32)])
                tot_q = plsc.load_gather(qvm, [iota16, jnp.zeros((_LANES,), jnp.int32)])
                for l in range(1, _LANES):
                    li = jnp.full((_LANES,), l, jnp.int32)
                    tot_s = tot_s + plsc.load_gather(svm, [iota16, li])
                    tot_q = tot_q + plsc.load_gather(qvm, [iota16, li])
                mean16 = tot_s * inv_n
                var16 = tot_q * inv_n - mean16 * mean16
                istd16 = _rsqrt(var16 + _EPS)
                for j in range(_UNROLL):
                    t = t0 + j
                    jf = jnp.full((_LANES,), j, jnp.int32)
                    gm = _lane_shuffle(mean16, jf)
                    gi = _lane_shuffle(istd16, jf)
                    for k in range(n_sub):
                        x = ov[t, pl.ds(k * _LANES, _LANES)]
                        ov[t, pl.ds(k * _LANES, _LANES)] = (
                            (x - gm) * gi * gs[k] + bs[k])
                return carry

            lax.fori_loop(0, _CHUNK // _UNROLL, tok_body, 0)

        def start_writeback(ci, b):
            pltpu.make_async_copy(
                obs[b], out_hbm.at[pl.ds(base + ci * _CHUNK, _CHUNK)],
                sos[b]).start()

        start_gather(0, 0)

        def loop_body(ci2, carry):
            ci_a = ci2 * 2
            ci_b = ci_a + 1
            start_gather(ci_b, 1)
            wait_gather(ci_a, 0)

            @pl.when(ci2 > 0)
            def _():
                wait_writeback(0)

            pos_add(ci_a, 0)
            compute(ci_a, 0)
            start_writeback(ci_a, 0)

            @pl.when(ci2 < n2 - 1)
            def _():
                start_gather(ci_a + 2, 0)

            wait_gather(ci_b, 1)

            @pl.when(ci2 > 0)
            def _():
                wait_writeback(1)

            pos_add(ci_b, 1)
            compute(ci_b, 1)
            start_writeback(ci_b, 1)
            return carry

        lax.fori_loop(0, n2, loop_body, 0)
        wait_writeback(0)
        wait_writeback(1)

    return sc_kernel


@jax.jit
def kernel(input_ids, token_type_ids, word_embeddings, position_embeddings,
           token_type_embeddings, ln_gamma, ln_beta):
    bsz, seq = input_ids.shape
    vocab, emb = word_embeddings.shape
    n_tokens = bsz * seq
    per_w = n_tokens // _NW
    n_chunks = per_w // _CHUNK

    ids = input_ids.astype(jnp.int32).reshape(_NW, n_chunks, _CHUNK)
    tts = token_type_ids.astype(jnp.int32).reshape(_NW, n_chunks, _CHUNK)
    pos2 = position_embeddings[:seq] + token_type_embeddings[0][None, :]
    pos2 = jnp.concatenate([pos2, pos2[:_CHUNK]], axis=0)
    cst = jnp.stack(
        [ln_gamma, ln_beta,
         token_type_embeddings[1] - token_type_embeddings[0]])

    pix = jnp.remainder(
        jnp.arange(per_w, dtype=jnp.int32), seq).reshape(n_chunks, _CHUNK)
    sc = _make_sc_kernel(n_tokens, emb, seq)
    out = sc(ids, tts, word_embeddings, pos2, cst, pix)
    return out.reshape(bsz, seq, emb)


# speedup vs baseline: 1.1657x; 1.1657x over previous
import functools

import jax
import jax.numpy as jnp
from jax import lax
from jax.experimental import pallas as pl
from jax.experimental.pallas import tpu as pltpu
from jax.experimental.pallas import tpu_sc as plsc

_EPS = 1e-12
_NC = 2
_NS = 16
_NW = _NC * _NS
_CHUNK = 128
_TCBLK = 1600


def _make_sc_gather(n_tokens, emb):
    per_w = n_tokens // _NW
    n_chunks = per_w // _CHUNK
    n2 = n_chunks // 2
    mesh = plsc.VectorSubcoreMesh(core_axis_name="c", subcore_axis_name="s")

    @functools.partial(
        pl.kernel,
        mesh=mesh,
        compiler_params=pltpu.CompilerParams(needs_layout_passes=False),
        out_type=jax.ShapeDtypeStruct((n_tokens, emb), jnp.float32),
        scratch_types=[
            pltpu.VMEM((n_chunks, _CHUNK), jnp.int32),
            pltpu.VMEM((_CHUNK, emb), jnp.float32),
            pltpu.VMEM((_CHUNK, emb), jnp.float32),
            pltpu.SemaphoreType.DMA,
            pltpu.SemaphoreType.DMA,
            pltpu.SemaphoreType.DMA,
            pltpu.SemaphoreType.DMA,
        ],
    )
    def sc_kernel(wid_hbm, word_hbm, out_hbm,
                  ids_v, row0, row1, sw0, sw1, so0, so1):
        wid = lax.axis_index("s") * _NC + lax.axis_index("c")
        base = wid * per_w
        pltpu.sync_copy(wid_hbm.at[wid], ids_v)

        rows = (row0, row1)
        sws = (sw0, sw1)
        sos = (so0, so1)

        def start_gather(ci, b):
            pltpu.make_async_copy(
                word_hbm.at[ids_v.at[ci]], rows[b], sws[b]).start()

        def wait_gather(ci, b):
            pltpu.make_async_copy(
                word_hbm.at[ids_v.at[ci]], rows[b], sws[b]).wait()

        def start_writeback(ci, b):
            pltpu.make_async_copy(
                rows[b], out_hbm.at[pl.ds(base + ci * _CHUNK, _CHUNK)],
                sos[b]).start()

        def wait_writeback(b):
            pltpu.make_async_copy(
                rows[b], out_hbm.at[pl.ds(base, _CHUNK)], sos[b]).wait()

        start_gather(0, 0)

        def loop_body(ci2, carry):
            ci_a = ci2 * 2
            ci_b = ci_a + 1
            start_gather(ci_b, 1)
            wait_gather(ci_a, 0)

            @pl.when(ci2 > 0)
            def _():
                wait_writeback(0)

            start_writeback(ci_a, 0)

            @pl.when(ci2 < n2 - 1)
            def _():
                start_gather(ci_a + 2, 0)

            wait_gather(ci_b, 1)

            @pl.when(ci2 > 0)
            def _():
                wait_writeback(1)

            start_writeback(ci_b, 1)
            return carry

        lax.fori_loop(0, n2, loop_body, 0)
        wait_writeback(0)
        wait_writeback(1)

    return sc_kernel


def _tc_ln_body(x_ref, pos_ref, ttf_ref, cst_ref, o_ref):
    x = (x_ref[...] + pos_ref[...]
         + ttf_ref[...] * cst_ref[0, :][None, :])
    mean = jnp.mean(x, axis=1, keepdims=True)
    var = jnp.mean(x * x, axis=1, keepdims=True) - mean * mean
    inv = lax.rsqrt(var + _EPS)
    o_ref[...] = ((x - mean) * inv * cst_ref[1, :][None, :]
                  + cst_ref[2, :][None, :])


def _tc_ln(rows, posfull, ttf, cst, n_tokens, emb):
    grid = (n_tokens // _TCBLK,)
    return pl.pallas_call(
        _tc_ln_body,
        grid=grid,
        in_specs=[
            pl.BlockSpec((_TCBLK, emb), lambda b: (b, 0)),
            pl.BlockSpec((_TCBLK, emb), lambda b: (0, 0)),
            pl.BlockSpec((_TCBLK, 1), lambda b: (b, 0)),
            pl.BlockSpec((3, emb), lambda b: (0, 0)),
        ],
        out_specs=pl.BlockSpec((_TCBLK, emb), lambda b: (b, 0)),
        out_shape=jax.ShapeDtypeStruct((n_tokens, emb), jnp.float32),
    )(rows, posfull, ttf, cst)


@jax.jit
def kernel(input_ids, token_type_ids, word_embeddings, position_embeddings,
           token_type_embeddings, ln_gamma, ln_beta):
    bsz, seq = input_ids.shape
    vocab, emb = word_embeddings.shape
    n_tokens = bsz * seq
    per_w = n_tokens // _NW
    n_chunks = per_w // _CHUNK

    ids = input_ids.astype(jnp.int32).reshape(_NW, n_chunks, _CHUNK)
    pos2 = position_embeddings[:seq] + token_type_embeddings[0][None, :]
    posfull = jnp.tile(pos2, (_TCBLK // seq, 1))
    ttf = token_type_ids.astype(jnp.float32).reshape(n_tokens, 1)
    cst = jnp.stack(
        [token_type_embeddings[1] - token_type_embeddings[0],
         ln_gamma, ln_beta])

    sc = _make_sc_gather(n_tokens, emb)
    rows = sc(ids, word_embeddings)
    out = _tc_ln(rows, posfull, ttf, cst, n_tokens, emb)
    return out.reshape(bsz, seq, emb)
